# Initial kernel scaffold; baseline (speedup 1.0000x reference)
import jax
import jax.numpy as jnp
from jax.experimental import pallas as pl


def kernel(x, edge_index, edge_attr, W1, b1, W2, b2, Wf1, bf1, Wf2, bf2):
    raise NotImplementedError("write your pallas kernel here")


# speedup vs baseline: 13.3644x; 13.3644x over previous
import functools

import jax
import jax.numpy as jnp
from jax import lax
from jax.experimental import pallas as pl
from jax.experimental.pallas import tpu as pltpu
from jax.experimental.pallas import tpu_sc as plsc

N = 10000
E = 320000
D_IN = 128
H = 16
C = 16

NC = 2
NS = 16
NW = NC * NS
EPW = E // NW
CHUNK = 80
NCHUNK = EPW // CHUNK
NPAD = 10240
RPT = NPAD // NS
F32 = jnp.float32

_sc_mesh = plsc.VectorSubcoreMesh(core_axis_name="c", subcore_axis_name="s")
_sc_params = pltpu.CompilerParams(needs_layout_passes=False, use_tc_tiling_on_sc=False)


@functools.partial(
    pl.kernel,
    mesh=_sc_mesh,
    out_type=jax.ShapeDtypeStruct((NW, N), F32),
    compiler_params=_sc_params,
    scratch_types=[
        pltpu.VMEM((EPW,), jnp.int32),
        pltpu.VMEM((N,), F32),
    ],
)
def _sc_degree(dst_hbm, out_hbm, dstv, degv):
    wid = lax.axis_index("s") * NC + lax.axis_index("c")
    pltpu.sync_copy(dst_hbm.at[wid], dstv)
    zeros = jnp.zeros((16,), F32)
    ones = jnp.ones((16,), F32)

    def zbody(i, _):
        degv[pl.ds(i * 16, 16)] = zeros
        return 0

    lax.fori_loop(0, N // 16, zbody, 0)

    def body(i, _):
        idx = dstv[pl.ds(i * 16, 16)]
        plsc.addupdate_scatter(degv, [idx], ones)
        return 0

    lax.fori_loop(0, EPW // 16, body, 0)
    pltpu.sync_copy(degv, out_hbm.at[wid])


@functools.partial(
    pl.kernel,
    mesh=_sc_mesh,
    out_type=jax.ShapeDtypeStruct((NC, NPAD, H), F32),
    compiler_params=_sc_params,
    scratch_types=[
        pltpu.VMEM((NCHUNK, CHUNK), jnp.int32),
        pltpu.VMEM((NCHUNK, CHUNK), jnp.int32),
        pltpu.VMEM((CHUNK, H), F32),
        pltpu.VMEM((RPT, H), F32),
        pltpu.VMEM_SHARED((NPAD, H), F32),
        pltpu.SemaphoreType.DMA,
    ],
)
def _sc_message(y_hbm, src_hbm, dst_hbm, out_hbm, srcv, dstv, rows, slab, acc, sem):
    cid = lax.axis_index("c")
    sid = lax.axis_index("s")
    wid = sid * NC + cid
    zeros = jnp.zeros((16,), F32)

    def zbody(i, _):
        slab[i, :] = zeros
        return 0

    lax.fori_loop(0, RPT, zbody, 0)
    pltpu.sync_copy(slab, acc.at[pl.ds(sid * RPT, RPT)])
    plsc.subcore_barrier()

    pltpu.sync_copy(src_hbm.at[wid], srcv)
    pltpu.sync_copy(dst_hbm.at[wid], dstv)

    def body(j, _):
        pltpu.async_copy(y_hbm.at[srcv.at[j]], rows, sem).wait()
        pltpu.sync_copy(rows, acc.at[dstv.at[j]], add=True)
        return 0

    lax.fori_loop(0, NCHUNK, body, 0)
    plsc.subcore_barrier()

    pltpu.sync_copy(acc.at[pl.ds(sid * RPT, RPT)], slab)
    pltpu.sync_copy(slab, out_hbm.at[cid, pl.ds(sid * RPT, RPT)])


@functools.partial(
    pl.kernel,
    mesh=_sc_mesh,
    out_type=jax.ShapeDtypeStruct((E, H), F32),
    compiler_params=_sc_params,
    scratch_types=[
        pltpu.VMEM((NCHUNK, CHUNK), jnp.int32),
        pltpu.VMEM((NCHUNK, CHUNK), jnp.int32),
        pltpu.VMEM((CHUNK, H), F32),
        pltpu.VMEM((CHUNK, H), F32),
        pltpu.SemaphoreType.DMA,
        pltpu.SemaphoreType.DMA,
    ],
)
def _sc_edge_sum(a_hbm, b_hbm, src_hbm, dst_hbm, out_hbm, srcv, dstv, ra, rb, sa, sb):
    wid = lax.axis_index("s") * NC + lax.axis_index("c")
    base = wid * EPW
    pltpu.sync_copy(src_hbm.at[wid], srcv)
    pltpu.sync_copy(dst_hbm.at[wid], dstv)

    def body(j, _):
        ca = pltpu.async_copy(a_hbm.at[srcv.at[j]], ra, sa)
        cb = pltpu.async_copy(b_hbm.at[dstv.at[j]], rb, sb)
        ca.wait()
        cb.wait()

        def add_body(r, _):
            ra[r, :] = ra[r, :] + rb[r, :]
            return 0

        lax.fori_loop(0, CHUNK, add_body, 0)
        pltpu.sync_copy(ra, out_hbm.at[pl.ds(base + j * CHUNK, CHUNK)])
        return 0

    lax.fori_loop(0, NCHUNK, body, 0)


def _tc_prep_body(x_ref, w1_ref, degp_ref, dinv_ref, y1_ref):
    xw = jnp.dot(x_ref[...], w1_ref[...], preferred_element_type=F32)
    deg = jnp.sum(degp_ref[...], axis=0) + 1.0
    d16 = jnp.broadcast_to(lax.rsqrt(deg)[:, None], (N, H))
    dinv_ref[...] = d16
    y1_ref[...] = d16 * xw


_tc_prep = pl.pallas_call(
    _tc_prep_body,
    out_shape=[jax.ShapeDtypeStruct((N, H), F32), jax.ShapeDtypeStruct((N, H), F32)],
)


def _tc_mid_body(mp_ref, y_ref, dinv_ref, b_ref, w_ref, out_ref):
    m = mp_ref[0, :N, :] + mp_ref[1, :N, :] + y_ref[...]
    h = jnp.maximum(dinv_ref[...] * m + b_ref[...], 0.0)
    out_ref[...] = dinv_ref[...] * jnp.dot(h, w_ref[...], preferred_element_type=F32)


_tc_mid = pl.pallas_call(
    _tc_mid_body,
    out_shape=jax.ShapeDtypeStruct((N, H), F32),
)


def _tc_post_body(mp_ref, y_ref, dinv_ref, b_ref, wf1_ref, bf1_ref, a_ref, bb_ref):
    m = mp_ref[0, :N, :] + mp_ref[1, :N, :] + y_ref[...]
    h = jnp.maximum(dinv_ref[...] * m + b_ref[...], 0.0)
    a_ref[...] = jnp.dot(h, wf1_ref[:H, :], preferred_element_type=F32) + bf1_ref[...]
    bb_ref[...] = jnp.dot(h, wf1_ref[H:, :], preferred_element_type=F32)


_tc_post = pl.pallas_call(
    _tc_post_body,
    out_shape=[jax.ShapeDtypeStruct((N, H), F32), jax.ShapeDtypeStruct((N, H), F32)],
)


_EBLK = 4000


def _tc_final_body(s_ref, wf2_ref, bf2_ref, out_ref):
    ef = jnp.maximum(s_ref[...], 0.0)
    z = jnp.dot(ef, wf2_ref[...], preferred_element_type=F32) + bf2_ref[...]
    z = z - jnp.max(z, axis=1, keepdims=True)
    out_ref[...] = z - jnp.log(jnp.sum(jnp.exp(z), axis=1, keepdims=True))


_tc_final = pl.pallas_call(
    _tc_final_body,
    grid=(E // _EBLK,),
    in_specs=[
        pl.BlockSpec((_EBLK, H), lambda i: (i, 0)),
        pl.BlockSpec((H, C), lambda i: (0, 0)),
        pl.BlockSpec((1, C), lambda i: (0, 0)),
    ],
    out_specs=pl.BlockSpec((_EBLK, C), lambda i: (i, 0)),
    out_shape=jax.ShapeDtypeStruct((E, C), F32),
)


def kernel(x, edge_index, edge_attr, W1, b1, W2, b2, Wf1, bf1, Wf2, bf2):
    src3 = edge_index[0].reshape(NW, NCHUNK, CHUNK)
    dst3 = edge_index[1].reshape(NW, NCHUNK, CHUNK)
    dst2 = edge_index[1].reshape(NW, EPW)

    degp = _sc_degree(dst2)
    dinv16, y1 = _tc_prep(x, W1, degp)
    m1 = _sc_message(y1, src3, dst3)
    y2 = _tc_mid(m1, y1, dinv16, b1.reshape(1, H), W2)
    m2 = _sc_message(y2, src3, dst3)
    A, B = _tc_post(m2, y2, dinv16, b2.reshape(1, H), Wf1, bf1.reshape(1, H))
    S = _sc_edge_sum(A, B, src3, dst3)
    return _tc_final(S, Wf2, bf2.reshape(1, C))
